# Initial kernel scaffold; baseline (speedup 1.0000x reference)
import jax
import jax.numpy as jnp
from jax.experimental import pallas as pl


def kernel(inputs, table, W1, b1, W2, b2, W3, b3):
    raise NotImplementedError("write your pallas kernel here")


# speedup vs baseline: 2.2601x; 2.2601x over previous
import functools

import jax
import jax.numpy as jnp
from jax import lax
from jax.experimental import pallas as pl
from jax.experimental.pallas import tpu as pltpu
from jax.experimental.pallas import tpu_sc as plsc

_V, _E, _H, _O = 1000000, 64, 128, 5
_B, _L = 16384, 50
_OP = 16
_BLK = 8000

_NC, _NS = 2, 16
_NW = _NC * _NS
_BPW = _B // _NW
_CB = 32
_NCH = _BPW // _CB
_ICH = _CB * _L
_G = 80
_NG = _ICH // _G


def _mlp_body(x_ref, w1_ref, b1_ref, w2_ref, b2_ref, w3_ref, b3_ref, o_ref):
    h = jnp.tanh(jnp.dot(x_ref[...], w1_ref[...],
                         preferred_element_type=jnp.float32) + b1_ref[...])
    h = jnp.tanh(jnp.dot(h, w2_ref[...],
                         preferred_element_type=jnp.float32) + b2_ref[...])
    o_ref[...] = jnp.dot(h, w3_ref[...],
                         preferred_element_type=jnp.float32) + b3_ref[...]


def _precompute_logits(table, W1, b1, W2, b2, W3p, b3p):
    return pl.pallas_call(
        _mlp_body,
        grid=(_V // _BLK,),
        in_specs=[
            pl.BlockSpec((_BLK, _E), lambda i: (i, 0)),
            pl.BlockSpec((_E, _H), lambda i: (0, 0)),
            pl.BlockSpec((1, _H), lambda i: (0, 0)),
            pl.BlockSpec((_H, _H), lambda i: (0, 0)),
            pl.BlockSpec((1, _H), lambda i: (0, 0)),
            pl.BlockSpec((_H, _OP), lambda i: (0, 0)),
            pl.BlockSpec((1, _OP), lambda i: (0, 0)),
        ],
        out_specs=pl.BlockSpec((_BLK, _OP), lambda i: (i, 0)),
        out_shape=jax.ShapeDtypeStruct((_V, _OP), jnp.float32),
    )(table, W1, b1, W2, b2, W3p, b3p)


def _sc_gather_sum(p, idx_flat):
    mesh = plsc.VectorSubcoreMesh(core_axis_name="c", subcore_axis_name="s",
                                  num_cores=_NC, num_subcores=_NS)

    @functools.partial(
        pl.kernel,
        mesh=mesh,
        compiler_params=pltpu.CompilerParams(use_tc_tiling_on_sc=False),
        out_type=jax.ShapeDtypeStruct((_B, _OP), jnp.float32),
        scratch_types=[
            pltpu.VMEM((_ICH,), jnp.int32),
            pltpu.VMEM((_ICH, _OP), jnp.float32),
            pltpu.VMEM((_CB, _OP), jnp.float32),
            pltpu.SemaphoreType.DMA,
        ],
    )
    def k(p_hbm, idx_hbm, out_hbm, idx_v, rows_v, out_v, sem):
        wid = lax.axis_index("s") * _NC + lax.axis_index("c")

        def chunk(ch, carry):
            ibase = pl.multiple_of((wid * _NCH + ch) * _ICH, _ICH)
            pltpu.sync_copy(idx_hbm.at[pl.ds(ibase, _ICH)], idx_v)
            descs = [
                pltpu.async_copy(p_hbm.at[idx_v.at[pl.ds(g * _G, _G)]],
                                 rows_v.at[pl.ds(g * _G, _G)], sem)
                for g in range(_NG)
            ]
            for d in descs:
                d.wait()

            def row(r, c2):
                acc = rows_v[r * _L]
                for l in range(1, _L):
                    acc = acc + rows_v[r * _L + l]
                out_v[r] = acc
                return c2

            lax.fori_loop(0, _CB, row, 0)
            obase = pl.multiple_of(wid * _BPW + ch * _CB, _CB)
            pltpu.sync_copy(out_v, out_hbm.at[pl.ds(obase, _CB)])
            return carry

        lax.fori_loop(0, _NCH, chunk, 0)

    return k(p, idx_flat)


def kernel(inputs, table, W1, b1, W2, b2, W3, b3):
    W3p = jnp.pad(W3, ((0, 0), (0, _OP - _O)))
    b3p = jnp.pad(b3, (0, _OP - _O)).reshape(1, _OP)
    p = _precompute_logits(table, W1, b1.reshape(1, _H), W2,
                           b2.reshape(1, _H), W3p, b3p)
    out = _sc_gather_sum(p, inputs.reshape(_B * _L))
    return out[:, :_O]


# speedup vs baseline: 3.3547x; 1.4843x over previous
import functools

import jax
import jax.numpy as jnp
from jax import lax
from jax.experimental import pallas as pl
from jax.experimental.pallas import tpu as pltpu
from jax.experimental.pallas import tpu_sc as plsc

_V, _E, _H, _O = 1000000, 64, 128, 5
_B, _L = 16384, 50
_OP = 16
_BLK = 40000

_NC, _NS = 2, 16
_NW = _NC * _NS
_BPW = _B // _NW
_CB = 32
_NCH = _BPW // _CB
_ICH = _CB * _L
_G = 80
_NG = _ICH // _G


_BLK8 = _BLK // 8


def _mlp_body(x_ref, w1_ref, b1_ref, w2_ref, b2_ref, w3s_ref, b3r_ref, o_ref):
    bm = lambda a, b: jnp.dot(a.astype(jnp.bfloat16), b.astype(jnp.bfloat16),
                              preferred_element_type=jnp.float32)
    acc = b3r_ref[...]
    for a in range(8):
        h = jnp.tanh(bm(x_ref[a], w1_ref[...]) + b1_ref[...])
        h = jnp.tanh(bm(h, w2_ref[...]) + b2_ref[...])
        acc = acc + bm(h, w3s_ref[a])
    o_ref[...] = acc


def _precompute_logits(table_r, W1, b1, W2, b2, W3s, b3row):
    return pl.pallas_call(
        _mlp_body,
        grid=(_V // _BLK,),
        in_specs=[
            pl.BlockSpec((8, _BLK8, _E), lambda i: (0, i, 0)),
            pl.BlockSpec((_E, _H), lambda i: (0, 0)),
            pl.BlockSpec((1, _H), lambda i: (0, 0)),
            pl.BlockSpec((_H, _H), lambda i: (0, 0)),
            pl.BlockSpec((1, _H), lambda i: (0, 0)),
            pl.BlockSpec((8, _H, 128), lambda i: (0, 0, 0)),
            pl.BlockSpec((1, 128), lambda i: (0, 0)),
        ],
        out_specs=pl.BlockSpec((_BLK8, 128), lambda i: (i, 0)),
        out_shape=jax.ShapeDtypeStruct((_V // 8, 128), jnp.float32),
    )(table_r, W1, b1, W2, b2, W3s, b3row)


def _sc_gather_sum(p, idx_flat):
    mesh = plsc.VectorSubcoreMesh(core_axis_name="c", subcore_axis_name="s",
                                  num_cores=_NC, num_subcores=_NS)

    @functools.partial(
        pl.kernel,
        mesh=mesh,
        compiler_params=pltpu.CompilerParams(use_tc_tiling_on_sc=False),
        out_type=jax.ShapeDtypeStruct((_B, _OP), jnp.float32),
        scratch_types=[
            pltpu.VMEM((_ICH,), jnp.int32),
            pltpu.VMEM((_ICH, _OP), jnp.float32),
            pltpu.VMEM((_CB, _OP), jnp.float32),
            pltpu.SemaphoreType.DMA,
        ],
    )
    def k(p_hbm, idx_hbm, out_hbm, idx_v, rows_v, out_v, sem):
        wid = lax.axis_index("s") * _NC + lax.axis_index("c")

        def chunk(ch, carry):
            ibase = pl.multiple_of((wid * _NCH + ch) * _ICH, _ICH)
            pltpu.sync_copy(idx_hbm.at[pl.ds(ibase, _ICH)], idx_v)
            descs = [
                pltpu.async_copy(p_hbm.at[idx_v.at[pl.ds(g * _G, _G)]],
                                 rows_v.at[pl.ds(g * _G, _G)], sem)
                for g in range(_NG)
            ]
            for d in descs:
                d.wait()

            def row(r, c2):
                acc = rows_v[r * _L]
                for l in range(1, _L):
                    acc = acc + rows_v[r * _L + l]
                out_v[r] = acc
                return c2

            lax.fori_loop(0, _CB, row, 0)
            obase = pl.multiple_of(wid * _BPW + ch * _CB, _CB)
            pltpu.sync_copy(out_v, out_hbm.at[pl.ds(obase, _CB)])
            return carry

        lax.fori_loop(0, _NCH, chunk, 0)

    return k(p, idx_flat)


def kernel(inputs, table, W1, b1, W2, b2, W3, b3):
    W3p = jnp.pad(W3, ((0, 0), (0, _OP - _O)))
    W3s = jnp.zeros((8, _H, 128), jnp.float32)
    for a in range(8):
        W3s = W3s.at[a, :, 16 * a:16 * (a + 1)].set(W3p)
    b3row = jnp.tile(jnp.pad(b3, (0, _OP - _O)), 8).reshape(1, 128)
    p2 = _precompute_logits(table.reshape(8, _V // 8, _E), W1,
                            b1.reshape(1, _H), W2, b2.reshape(1, _H),
                            W3s, b3row)
    idx = inputs.reshape(_B * _L)
    idx2 = (idx % (_V // 8)) * 8 + idx // (_V // 8)
    out = _sc_gather_sum(p2.reshape(_V, _OP), idx2)
    return out[:, :_O]


# speedup vs baseline: 3.7842x; 1.1280x over previous
import functools

import jax
import jax.numpy as jnp
from jax import lax
from jax.experimental import pallas as pl
from jax.experimental.pallas import tpu as pltpu
from jax.experimental.pallas import tpu_sc as plsc

_V, _E, _H, _O = 1000000, 64, 128, 5
_B, _L = 16384, 50
_OP = 16
_BLK = 8000
_BLK8 = _BLK // 8

_NC, _NS = 2, 16
_NW = _NC * _NS
_BPW = _B // _NW
_IPW = _BPW * _L
_CB = 32
_NCH = _BPW // _CB
_ICH = _CB * _L
_G = 80
_NG = _ICH // _G


def _mlp_body(x_ref, w1_ref, b1_ref, w2_ref, b2_ref, w3c_ref, b3r_ref, o_ref):
    bm = lambda a, b: jnp.dot(a.astype(jnp.bfloat16), b.astype(jnp.bfloat16),
                              preferred_element_type=jnp.float32)
    hs = []
    for a in range(8):
        h = jnp.tanh(bm(x_ref[a], w1_ref[...]) + b1_ref[...])
        h = jnp.tanh(bm(h, w2_ref[...]) + b2_ref[...])
        hs.append(h)
    hcat = jnp.concatenate(hs, axis=1)
    o_ref[...] = bm(hcat, w3c_ref[...]) + b3r_ref[...]


def _precompute_logits(table_r, W1, b1, W2, b2, W3c, b3row):
    return pl.pallas_call(
        _mlp_body,
        grid=(_V // _BLK,),
        in_specs=[
            pl.BlockSpec((8, _BLK8, _E), lambda i: (0, i, 0)),
            pl.BlockSpec((_E, _H), lambda i: (0, 0)),
            pl.BlockSpec((1, _H), lambda i: (0, 0)),
            pl.BlockSpec((_H, _H), lambda i: (0, 0)),
            pl.BlockSpec((1, _H), lambda i: (0, 0)),
            pl.BlockSpec((8 * _H, 128), lambda i: (0, 0)),
            pl.BlockSpec((1, 128), lambda i: (0, 0)),
        ],
        out_specs=pl.BlockSpec((_BLK8, 128), lambda i: (i, 0)),
        out_shape=jax.ShapeDtypeStruct((_V // 8, 128), jnp.float32),
    )(table_r, W1, b1, W2, b2, W3c, b3row)


def _sc_gather_sum(p, idx_flat):
    mesh = plsc.VectorSubcoreMesh(core_axis_name="c", subcore_axis_name="s",
                                  num_cores=_NC, num_subcores=_NS)

    @functools.partial(
        pl.kernel,
        mesh=mesh,
        compiler_params=pltpu.CompilerParams(use_tc_tiling_on_sc=False),
        out_type=jax.ShapeDtypeStruct((_B, _OP), jnp.float32),
        scratch_types=[
            pltpu.VMEM((_IPW,), jnp.int32),
            pltpu.VMEM((2 * _ICH, _OP), jnp.float32),
            pltpu.VMEM((_CB, _OP), jnp.float32),
            pltpu.SemaphoreType.DMA,
            pltpu.SemaphoreType.DMA,
        ],
    )
    def k(p_hbm, idx_hbm, out_hbm, idx_v, rows_v, out_v, sem0, sem1):
        wid = lax.axis_index("s") * _NC + lax.axis_index("c")
        pltpu.sync_copy(
            idx_hbm.at[pl.ds(pl.multiple_of(wid * _IPW, _IPW), _IPW)], idx_v)
        sems = (sem0, sem1)

        def issue(ch, half):
            cbase = pl.multiple_of(ch * _ICH, _ICH)
            for g in range(_NG):
                pltpu.async_copy(
                    p_hbm.at[idx_v.at[pl.ds(cbase + g * _G, _G)]],
                    rows_v.at[pl.ds(half * _ICH + g * _G, _G)], sems[half])

        def drain(half):
            pltpu.make_async_copy(
                p_hbm.at[pl.ds(0, _ICH)],
                rows_v.at[pl.ds(half * _ICH, _ICH)], sems[half]).wait()

        def accum(ch, half):
            def row(r, c2):
                base = half * _ICH + r * _L
                vals = [rows_v[base + l] for l in range(_L)]
                while len(vals) > 1:
                    nxt = [vals[i] + vals[i + 1]
                           for i in range(0, len(vals) - 1, 2)]
                    if len(vals) % 2:
                        nxt.append(vals[-1])
                    vals = nxt
                out_v[r] = vals[0]
                return c2

            lax.fori_loop(0, _CB, row, 0)
            obase = pl.multiple_of(wid * _BPW + ch * _CB, _CB)
            pltpu.sync_copy(out_v, out_hbm.at[pl.ds(obase, _CB)])

        issue(0, 0)

        def body(i, carry):
            ch0 = 2 * i
            issue(ch0 + 1, 1)
            drain(0)
            accum(ch0, 0)

            @pl.when(i < _NCH // 2 - 1)
            def _():
                issue(ch0 + 2, 0)

            drain(1)
            accum(ch0 + 1, 1)
            return carry

        lax.fori_loop(0, _NCH // 2, body, 0)

    return k(p, idx_flat)


def kernel(inputs, table, W1, b1, W2, b2, W3, b3):
    W3p = jnp.pad(W3, ((0, 0), (0, _OP - _O)))
    W3c = jnp.zeros((8 * _H, 128), jnp.float32)
    for a in range(8):
        W3c = W3c.at[_H * a:_H * (a + 1), 16 * a:16 * (a + 1)].set(W3p)
    b3row = jnp.tile(jnp.pad(b3, (0, _OP - _O)), 8).reshape(1, 128)
    p2 = _precompute_logits(table.reshape(8, _V // 8, _E), W1,
                            b1.reshape(1, _H), W2, b2.reshape(1, _H),
                            W3c, b3row)
    idx = inputs.reshape(_B * _L)
    idx2 = (idx % (_V // 8)) * 8 + idx // (_V // 8)
    out = _sc_gather_sum(p2.reshape(_V, _OP), idx2)
    return out[:, :_O]


# speedup vs baseline: 3.7911x; 1.0018x over previous
import functools

import jax
import jax.numpy as jnp
from jax import lax
from jax.experimental import pallas as pl
from jax.experimental.pallas import tpu as pltpu
from jax.experimental.pallas import tpu_sc as plsc

_V, _E, _H, _O = 1000000, 64, 128, 5
_B, _L = 16384, 50
_OP = 16
_BLK = 8000
_BLK8 = _BLK // 8

_NC, _NS = 2, 16
_NW = _NC * _NS
_BPW = _B // _NW
_IPW = _BPW * _L
_CB = 32
_NCH = _BPW // _CB
_ICH = _CB * _L
_G = 80
_NG = _ICH // _G


def _mlp_body(x_ref, w1_ref, b1_ref, w2_ref, b2_ref, w3c_ref, b3r_ref, o_ref):
    bm = lambda a, b: jnp.dot(a.astype(jnp.bfloat16), b.astype(jnp.bfloat16),
                              preferred_element_type=jnp.float32)
    hs = []
    for a in range(8):
        h = jnp.tanh(bm(x_ref[a], w1_ref[...]) + b1_ref[...])
        h = jnp.tanh(bm(h, w2_ref[...]) + b2_ref[...])
        hs.append(h)
    hcat = jnp.concatenate(hs, axis=1)
    o_ref[...] = (bm(hcat, w3c_ref[...]) + b3r_ref[...]).reshape(_BLK8 * 128)


def _precompute_logits(table_r, W1, b1, W2, b2, W3c, b3row):
    return pl.pallas_call(
        _mlp_body,
        grid=(_V // _BLK,),
        in_specs=[
            pl.BlockSpec((8, _BLK8, _E), lambda i: (0, i, 0)),
            pl.BlockSpec((_E, _H), lambda i: (0, 0)),
            pl.BlockSpec((1, _H), lambda i: (0, 0)),
            pl.BlockSpec((_H, _H), lambda i: (0, 0)),
            pl.BlockSpec((1, _H), lambda i: (0, 0)),
            pl.BlockSpec((8 * _H, 128), lambda i: (0, 0)),
            pl.BlockSpec((1, 128), lambda i: (0, 0)),
        ],
        out_specs=pl.BlockSpec((_BLK8 * 128,), lambda i: (i,)),
        out_shape=jax.ShapeDtypeStruct((_V * _OP,), jnp.float32),
    )(table_r, W1, b1, W2, b2, W3c, b3row)


def _sc_gather_sum(p, idx_flat):
    mesh = plsc.VectorSubcoreMesh(core_axis_name="c", subcore_axis_name="s",
                                  num_cores=_NC, num_subcores=_NS)

    @functools.partial(
        pl.kernel,
        mesh=mesh,
        compiler_params=pltpu.CompilerParams(use_tc_tiling_on_sc=False),
        out_type=jax.ShapeDtypeStruct((_B, _OP), jnp.float32),
        scratch_types=[
            pltpu.VMEM((_IPW,), jnp.int32),
            pltpu.VMEM((2 * _ICH, _OP), jnp.float32),
            pltpu.VMEM((_CB, _OP), jnp.float32),
            pltpu.SemaphoreType.DMA,
            pltpu.SemaphoreType.DMA,
        ],
    )
    def k(p_hbm, idx_hbm, out_hbm, idx_v, rows_v, out_v, sem0, sem1):
        wid = lax.axis_index("s") * _NC + lax.axis_index("c")
        pltpu.sync_copy(
            idx_hbm.at[pl.ds(pl.multiple_of(wid * _IPW, _IPW), _IPW)], idx_v)
        sems = (sem0, sem1)

        def issue(ch, half):
            cbase = pl.multiple_of(ch * _ICH, _ICH)
            for g in range(_NG):
                pltpu.async_copy(
                    p_hbm.at[idx_v.at[pl.ds(cbase + g * _G, _G)]],
                    rows_v.at[pl.ds(half * _ICH + g * _G, _G)], sems[half])

        def drain(half):
            pltpu.make_async_copy(
                p_hbm.at[pl.ds(0, _ICH)],
                rows_v.at[pl.ds(half * _ICH, _ICH)], sems[half]).wait()

        def accum(ch, half):
            def row(r, c2):
                base = half * _ICH + r * _L
                vals = [rows_v[base + l] for l in range(_L)]
                while len(vals) > 1:
                    nxt = [vals[i] + vals[i + 1]
                           for i in range(0, len(vals) - 1, 2)]
                    if len(vals) % 2:
                        nxt.append(vals[-1])
                    vals = nxt
                out_v[r] = vals[0]
                return c2

            lax.fori_loop(0, _CB, row, 0)
            obase = pl.multiple_of(wid * _BPW + ch * _CB, _CB)
            pltpu.sync_copy(out_v, out_hbm.at[pl.ds(obase, _CB)])

        issue(0, 0)

        def body(i, carry):
            ch0 = 2 * i
            issue(ch0 + 1, 1)
            drain(0)
            accum(ch0, 0)

            @pl.when(i < _NCH // 2 - 1)
            def _():
                issue(ch0 + 2, 0)

            drain(1)
            accum(ch0 + 1, 1)
            return carry

        lax.fori_loop(0, _NCH // 2, body, 0)

    return k(p, idx_flat)


def kernel(inputs, table, W1, b1, W2, b2, W3, b3):
    W3p = jnp.pad(W3, ((0, 0), (0, _OP - _O)))
    W3c = jnp.zeros((8 * _H, 128), jnp.float32)
    for a in range(8):
        W3c = W3c.at[_H * a:_H * (a + 1), 16 * a:16 * (a + 1)].set(W3p)
    b3row = jnp.tile(jnp.pad(b3, (0, _OP - _O)), 8).reshape(1, 128)
    p2 = _precompute_logits(table.reshape(8, _V // 8, _E), W1,
                            b1.reshape(1, _H), W2, b2.reshape(1, _H),
                            W3c, b3row)
    idx = inputs.reshape(_B * _L)
    idx2 = (idx % (_V // 8)) * 8 + idx // (_V // 8)
    out = _sc_gather_sum(p2.reshape(_V, _OP), idx2)
    return out[:, :_O]
